# Initial kernel scaffold; baseline (speedup 1.0000x reference)
import jax
import jax.numpy as jnp
from jax.experimental import pallas as pl


def kernel(reconstruction, original):
    raise NotImplementedError("write your pallas kernel here")


# speedup vs baseline: 1.4936x; 1.4936x over previous
import functools
import jax
import jax.numpy as jnp
from jax import lax
from jax.experimental import pallas as pl
from jax.experimental.pallas import tpu as pltpu
from jax.experimental.pallas import tpu_sc as plsc

_SIG_WEIGHT = 30.0
_CLOSE_MIN = 0.05

_W = 224
_ROWS = 448
_NW = 32
_ROWS_PER_W = _ROWS // _NW
_WORDS_PER_W = _ROWS_PER_W * _W
_CHUNKS = _WORDS_PER_W // 16
_PAD = 8


def _sc_body(r_hbm, a_hbm, out_hbm, a_v, r_v, acc_v, sem):
    c = lax.axis_index("c")
    s = lax.axis_index("s")
    wid = s * 2 + c
    base = wid * _WORDS_PER_W
    pltpu.sync_copy(a_hbm.at[pl.ds(base, _WORDS_PER_W)],
                    a_v.at[pl.ds(_PAD, _WORDS_PER_W)])
    pltpu.sync_copy(r_hbm.at[pl.ds(base, _WORDS_PER_W)], r_v)

    lane = lax.iota(jnp.int32, 16)
    zero = jnp.zeros((16,), jnp.float32)

    def chunk(k, acc):
        off = k * 16
        cpos = lax.rem(k, _W // 16)
        col = lane + cpos * 16
        a = a_v[pl.ds(_PAD + off, 16)]
        ap = a_v[pl.ds(_PAD + off - 1, 16)]
        an = a_v[pl.ds(_PAD + off + 1, 16)]
        r = r_v[pl.ds(off, 16)]
        onz = a != 0.0
        valid_next = (col < _W - 1) & (an != 0.0)
        valid_self = (col >= 1) & onz
        valid_prev = (col >= 2) & (ap != 0.0)
        m = jnp.where(valid_next, an,
                      jnp.where(valid_self, a,
                                jnp.where(valid_prev, ap, a)))
        d0 = r - a
        orig_mse = d0 * d0
        dm = r - m
        alt = dm * dm * dm + _CLOSE_MIN
        loss = jnp.minimum(orig_mse, alt)
        loss = jnp.where(onz, loss * _SIG_WEIGHT, loss)
        return acc + loss

    acc = lax.fori_loop(0, _CHUNKS, chunk, zero)
    acc_v[...] = acc
    pltpu.sync_copy(acc_v, out_hbm.at[wid])


def kernel(reconstruction, original):
    r_flat = reconstruction.reshape(_ROWS * _W)
    a_flat = original.reshape(_ROWS * _W)
    mesh = plsc.VectorSubcoreMesh(core_axis_name="c", subcore_axis_name="s")
    fn = functools.partial(
        pl.kernel, mesh=mesh,
        out_type=jax.ShapeDtypeStruct((_NW, 16), jnp.float32),
        scratch_types=[
            pltpu.VMEM((_WORDS_PER_W + 2 * _PAD,), jnp.float32),
            pltpu.VMEM((_WORDS_PER_W,), jnp.float32),
            pltpu.VMEM((16,), jnp.float32),
            pltpu.SemaphoreType.DMA,
        ],
    )(_sc_body)
    partials = fn(r_flat, a_flat)
    return jnp.sum(partials) / (_ROWS * _W)


# speedup vs baseline: 15.5162x; 10.3884x over previous
import jax
import jax.numpy as jnp
from jax.experimental import pallas as pl
from jax.experimental.pallas import tpu as pltpu

_FURTHEST = 1
_SIG_WEIGHT = 30.0
_CLOSE_MIN = 0.05


def _loss_kernel(r_ref, a_ref, o_ref):
    a = a_ref[...]
    r = r_ref[...]
    W = a.shape[1]
    inv_n = 1.0 / (a.shape[0] * a.shape[1])
    onz = a != 0.0
    col = jax.lax.broadcasted_iota(jnp.int32, a.shape, 1)
    a_next = jnp.concatenate([a[:, 1:], a[:, :1]], axis=1)
    a_prev = jnp.concatenate([a[:, -1:], a[:, :-1]], axis=1)
    valid_next = (col < W - 1) & (a_next != 0.0)
    valid_self = (col >= 1) & onz
    valid_prev = (col >= 2) & (a_prev != 0.0)
    m = jnp.where(valid_next, a_next,
                  jnp.where(valid_self, a,
                            jnp.where(valid_prev, a_prev, a)))
    d0 = r - a
    orig_mse = d0 * d0
    dm = r - m
    alt = dm * dm * dm + _CLOSE_MIN
    loss = jnp.minimum(orig_mse, alt)
    loss = jnp.where(onz, loss * _SIG_WEIGHT, loss)
    o_ref[0, 0] = jnp.sum(loss) * inv_n


def kernel(reconstruction, original):
    B, C, H, W = original.shape
    r2 = reconstruction.reshape(B * C * H, W)
    a2 = original.reshape(B * C * H, W)
    out = pl.pallas_call(
        _loss_kernel,
        out_shape=jax.ShapeDtypeStruct((1, 1), jnp.float32),
        out_specs=pl.BlockSpec(memory_space=pltpu.SMEM),
    )(r2, a2)
    return out[0, 0]
